# speedup vs baseline: 3.3273x; 3.3273x over previous
import functools

import jax
import jax.numpy as jnp
from jax import lax
from jax.experimental import pallas as pl
from jax.experimental.pallas import tpu as pltpu
from jax.experimental.pallas import tpu_sc as plsc

D_MODEL = 128
NUM_WORKERS = 32
CHUNK = 800


@functools.partial(jax.jit, static_argnames=("b_per_w", "n_chunks"))
def _sc_gather(x_flat, table, b_per_w, n_chunks):
    mesh = plsc.VectorSubcoreMesh(core_axis_name="c", subcore_axis_name="s")
    total = x_flat.shape[0]

    @functools.partial(
        pl.kernel,
        out_type=jax.ShapeDtypeStruct((total, D_MODEL), jnp.float32),
        mesh=mesh,
        scratch_types=[
            pltpu.VMEM((b_per_w,), jnp.int32),
            pltpu.VMEM((CHUNK, D_MODEL), jnp.float32),
            pltpu.SemaphoreType.DMA,
        ],
    )
    def k(x_hbm, tbl_hbm, out_hbm, idx_v, rows_v, sem):
        wid = lax.axis_index("s") * 2 + lax.axis_index("c")
        base = wid * b_per_w
        pltpu.sync_copy(x_hbm.at[pl.ds(base, b_per_w)], idx_v)

        def step(i, carry):
            off = i * CHUNK
            pltpu.async_copy(
                tbl_hbm.at[idx_v.at[pl.ds(off, CHUNK)]], rows_v, sem
            ).wait()
            pltpu.sync_copy(rows_v, out_hbm.at[pl.ds(base + off, CHUNK)])
            return carry

        lax.fori_loop(0, n_chunks, step, 0)

    return k(x_flat, table)


def kernel(x, table):
    n, s = x.shape
    total = n * s
    b_per_w = total // NUM_WORKERS
    n_chunks = b_per_w // CHUNK
    x_flat = x.reshape(total).astype(jnp.int32)
    out = _sc_gather(x_flat, table, b_per_w, n_chunks)
    return out.reshape(n, s, D_MODEL)
